# baseline (speedup 1.0000x reference)
import functools

import jax
import jax.numpy as jnp
from jax import lax
from jax.experimental import pallas as pl
from jax.experimental.pallas import tpu as pltpu
from jax.experimental.pallas import tpu_sc as plsc

N = 325
NROWPAD = 328
NPAD = 384
TOPK = 16
LANES = 16
NCH = NPAD // LANES

NW = 32
R_TOTAL = 96 * NROWPAD
BATCH = 24
NB_PER_W = R_TOTAL // (BATCH * NW)


def _tc_graph_body(x_ref, w_ref, o_ref):
    x = x_ref[0]
    a1 = jnp.dot(x, w_ref[0], preferred_element_type=jnp.float32)
    a2 = jnp.dot(x, w_ref[1], preferred_element_type=jnp.float32)
    g = lax.dot_general(a1, a2, (((1,), (1,)), ((), ())),
                        preferred_element_type=jnp.float32)
    g = jnp.maximum(g, 0.0)
    o_ref[...] = jnp.pad(g, ((0, NROWPAD - N), (0, NPAD - N)))


def _sc_body(g_hbm, out_hbm, in_v, out_v):
    wid = lax.axis_index("s") * 2 + lax.axis_index("c")
    base = wid * NB_PER_W * BATCH

    def batch_body(bi, carry):
        rb = base + bi * BATCH
        pltpu.sync_copy(g_hbm.at[pl.ds(rb, BATCH)], in_v)

        def row_body(r, c2):
            chunks = [in_v[r, pl.ds(c * LANES, LANES)] for c in range(NCH)]
            m = chunks[0]
            for c in range(1, NCH):
                m = jnp.maximum(m, chunks[c])
            mx = jnp.max(m)
            top, _ = plsc.sort_key_val(chunks[0], chunks[0])
            for c in range(1, NCH):
                v = chunks[c]
                dsc, _ = plsc.sort_key_val(v, v, descending=True)
                cand = jnp.maximum(top, dsc)
                top, _ = plsc.sort_key_val(cand, cand)
            t = jnp.min(top)
            acc = jnp.zeros((LANES,), jnp.float32)
            es = []
            for c in range(NCH):
                v = chunks[c]
                keep = jnp.logical_and(v >= t, v > 0.0)
                e = jnp.where(keep, jnp.exp(v - mx), 0.0)
                acc = acc + e
                es.append(e)
            den = jnp.broadcast_to(jnp.sum(acc) + 1e-5, (LANES,))
            inv = jnp.ones((LANES,), jnp.float32) / den
            for c in range(NCH):
                out_v[r, pl.ds(c * LANES, LANES)] = es[c] * inv
            return c2

        lax.fori_loop(0, BATCH, row_body, 0)
        pltpu.sync_copy(out_v, out_hbm.at[pl.ds(rb, BATCH)])
        return carry

    lax.fori_loop(0, NB_PER_W, batch_body, 0)


_sc_topk_softmax = functools.partial(
    pl.kernel,
    out_type=jax.ShapeDtypeStruct((R_TOTAL, NPAD), jnp.float32),
    mesh=plsc.VectorSubcoreMesh(core_axis_name="c", subcore_axis_name="s"),
    scratch_types=[
        pltpu.VMEM((BATCH, NPAD), jnp.float32),
        pltpu.VMEM((BATCH, NPAD), jnp.float32),
    ],
    compiler_params=pltpu.CompilerParams(needs_layout_passes=False),
)(_sc_body)


def kernel(c_input, weight):
    b, t, n, d = c_input.shape
    xs = c_input.reshape(b * t, n, d)
    g = pl.pallas_call(
        _tc_graph_body,
        grid=(b * t,),
        in_specs=[
            pl.BlockSpec((1, n, d), lambda i: (i, 0, 0)),
            pl.BlockSpec((2, d, weight.shape[2]), lambda i: (0, 0, 0)),
        ],
        out_specs=pl.BlockSpec((NROWPAD, NPAD), lambda i: (i, 0)),
        out_shape=jax.ShapeDtypeStruct((R_TOTAL, NPAD), jnp.float32),
    )(xs, weight)
    out = _sc_topk_softmax(g)
    return out.reshape(b * t, NROWPAD, NPAD)[:, :n, :n].reshape(b, t, n, n)


# speedup vs baseline: 1.2073x; 1.2073x over previous
import functools

import jax
import jax.numpy as jnp
from jax import lax
from jax.experimental import pallas as pl
from jax.experimental.pallas import tpu as pltpu
from jax.experimental.pallas import tpu_sc as plsc

N = 325
NROWPAD = 328
NPAD = 336
TOPK = 16
LANES = 16
NCH = NPAD // LANES

NW = 32
R_TOTAL = 96 * NROWPAD
BATCH = 24
NB_PER_W = R_TOTAL // (BATCH * NW)


def _tc_graph_body(x_ref, w_ref, o_ref):
    x = x_ref[0]
    a1 = jnp.dot(x, w_ref[0], preferred_element_type=jnp.float32)
    a2 = jnp.dot(x, w_ref[1], preferred_element_type=jnp.float32)
    g = lax.dot_general(a1, a2, (((1,), (1,)), ((), ())),
                        preferred_element_type=jnp.float32)
    g = jnp.maximum(g, 0.0)
    o_ref[...] = jnp.pad(g, ((0, NROWPAD - N), (0, NPAD - N)))


def _sc_body(g_hbm, out_hbm, in_a, in_b, out_v, sem_a, sem_b):
    wid = lax.axis_index("s") * 2 + lax.axis_index("c")
    base = wid * NB_PER_W * BATCH

    def _start_in(bi, buf, sem):
        pltpu.make_async_copy(
            g_hbm.at[pl.ds(base + bi * BATCH, BATCH)], buf, sem).start()

    def _wait_in(bi, buf, sem):
        pltpu.make_async_copy(
            g_hbm.at[pl.ds(base + bi * BATCH, BATCH)], buf, sem).wait()

    def _compute(bi, in_v):
        rb = base + bi * BATCH

        def row_body(r, c2):
            chunks = [in_v[r, pl.ds(c * LANES, LANES)] for c in range(NCH)]
            m = chunks[0]
            for c in range(1, NCH):
                m = jnp.maximum(m, chunks[c])
            mx = jnp.max(m)
            top, _ = plsc.sort_key_val(chunks[0], chunks[0])
            for c in range(1, NCH):
                v = chunks[c]
                dsc, _ = plsc.sort_key_val(v, v, descending=True)
                cand = jnp.maximum(top, dsc)
                top, _ = plsc.sort_key_val(cand, cand)
            t = jnp.min(top)
            acc = jnp.zeros((LANES,), jnp.float32)
            es = []
            for c in range(NCH):
                v = chunks[c]
                keep = jnp.logical_and(v >= t, v > 0.0)
                e = jnp.where(keep, jnp.exp(v - mx), 0.0)
                acc = acc + e
                es.append(e)
            den = jnp.broadcast_to(jnp.sum(acc) + 1e-5, (LANES,))
            inv = jnp.ones((LANES,), jnp.float32) / den
            for c in range(NCH):
                out_v[r, pl.ds(c * LANES, LANES)] = es[c] * inv
            return c2

        lax.fori_loop(0, BATCH, row_body, 0)
        pltpu.sync_copy(out_v, out_hbm.at[pl.ds(rb, BATCH)])

    _start_in(0, in_a, sem_a)

    def pair_body(i, carry):
        b0 = 2 * i
        _wait_in(b0, in_a, sem_a)
        _start_in(b0 + 1, in_b, sem_b)
        _compute(b0, in_a)
        _wait_in(b0 + 1, in_b, sem_b)

        @pl.when(b0 + 2 < NB_PER_W)
        def _():
            _start_in(b0 + 2, in_a, sem_a)
        _compute(b0 + 1, in_b)
        return carry

    lax.fori_loop(0, NB_PER_W // 2, pair_body, 0)
    _wait_in(NB_PER_W - 1, in_a, sem_a)
    _compute(NB_PER_W - 1, in_a)


_sc_topk_softmax = functools.partial(
    pl.kernel,
    out_type=jax.ShapeDtypeStruct((R_TOTAL, NPAD), jnp.float32),
    mesh=plsc.VectorSubcoreMesh(core_axis_name="c", subcore_axis_name="s"),
    scratch_types=[
        pltpu.VMEM((BATCH, NPAD), jnp.float32),
        pltpu.VMEM((BATCH, NPAD), jnp.float32),
        pltpu.VMEM((BATCH, NPAD), jnp.float32),
        pltpu.SemaphoreType.DMA,
        pltpu.SemaphoreType.DMA,
    ],
    compiler_params=pltpu.CompilerParams(needs_layout_passes=False),
)(_sc_body)


def kernel(c_input, weight):
    b, t, n, d = c_input.shape
    xs = c_input.reshape(b * t, n, d)
    g = pl.pallas_call(
        _tc_graph_body,
        grid=(b * t,),
        in_specs=[
            pl.BlockSpec((1, n, d), lambda i: (i, 0, 0)),
            pl.BlockSpec((2, d, weight.shape[2]), lambda i: (0, 0, 0)),
        ],
        out_specs=pl.BlockSpec((NROWPAD, NPAD), lambda i: (i, 0)),
        out_shape=jax.ShapeDtypeStruct((R_TOTAL, NPAD), jnp.float32),
    )(xs, weight)
    out = _sc_topk_softmax(g)
    return out.reshape(b * t, NROWPAD, NPAD)[:, :n, :n].reshape(b, t, n, n)


# speedup vs baseline: 1.2441x; 1.0305x over previous
import functools

import jax
import jax.numpy as jnp
from jax import lax
from jax.experimental import pallas as pl
from jax.experimental.pallas import tpu as pltpu
from jax.experimental.pallas import tpu_sc as plsc

N = 325
NROWPAD = 328
NPAD = 336
TOPK = 16
LANES = 16
NCH = NPAD // LANES

NW = 32
R_TOTAL = 96 * NROWPAD
BATCH = 24
NB_PER_W = R_TOTAL // (BATCH * NW)


def _tc_graph_body(x_ref, w_ref, o_ref):
    x = x_ref[0]
    a1 = jnp.dot(x, w_ref[0], preferred_element_type=jnp.float32)
    a2 = jnp.dot(x, w_ref[1], preferred_element_type=jnp.float32)
    g = lax.dot_general(a1, a2, (((1,), (1,)), ((), ())),
                        preferred_element_type=jnp.float32)
    g = jnp.maximum(g, 0.0)
    o_ref[...] = jnp.pad(g, ((0, NROWPAD - N), (0, NPAD - N)))


def _sc_body(g_hbm, out_hbm, in_a, in_b, out_a, out_b,
             sem_a, sem_b, osem_a, osem_b):
    wid = lax.axis_index("s") * 2 + lax.axis_index("c")
    base = wid * NB_PER_W * BATCH

    def _start_in(bi, buf, sem):
        pltpu.make_async_copy(
            g_hbm.at[pl.ds(base + bi * BATCH, BATCH)], buf, sem).start()

    def _wait_in(bi, buf, sem):
        pltpu.make_async_copy(
            g_hbm.at[pl.ds(base + bi * BATCH, BATCH)], buf, sem).wait()

    def _start_out(bi, buf, sem):
        pltpu.make_async_copy(
            buf, out_hbm.at[pl.ds(base + bi * BATCH, BATCH)], sem).start()

    def _wait_out(bi, buf, sem):
        pltpu.make_async_copy(
            buf, out_hbm.at[pl.ds(base + bi * BATCH, BATCH)], sem).wait()

    def _compute(bi, in_v, out_v):
        rb = base + bi * BATCH

        def row_body(r, c2):
            chunks = [in_v[r, pl.ds(c * LANES, LANES)] for c in range(NCH)]
            m = chunks[0]
            for c in range(1, NCH):
                m = jnp.maximum(m, chunks[c])
            mx = jnp.max(m)
            top, _ = plsc.sort_key_val(chunks[0], chunks[0])
            for c in range(1, NCH):
                v = chunks[c]
                dsc, _ = plsc.sort_key_val(v, v, descending=True)
                cand = jnp.maximum(top, dsc)
                top, _ = plsc.sort_key_val(cand, cand)
            t = jnp.min(top)
            acc = jnp.zeros((LANES,), jnp.float32)
            es = []
            for c in range(NCH):
                v = chunks[c]
                keep = jnp.logical_and(v >= t, v > 0.0)
                e = jnp.where(keep, jnp.exp(v - mx), 0.0)
                acc = acc + e
                es.append(e)
            den = jnp.broadcast_to(jnp.sum(acc) + 1e-5, (LANES,))
            inv = jnp.ones((LANES,), jnp.float32) / den
            for c in range(NCH):
                out_v[r, pl.ds(c * LANES, LANES)] = es[c] * inv
            return c2

        lax.fori_loop(0, BATCH, row_body, 0)

    _start_in(0, in_a, sem_a)

    def pair_body(i, carry):
        b0 = 2 * i
        _wait_in(b0, in_a, sem_a)
        _start_in(b0 + 1, in_b, sem_b)

        @pl.when(i > 0)
        def _():
            _wait_out(b0 - 2, out_a, osem_a)
        _compute(b0, in_a, out_a)
        _start_out(b0, out_a, osem_a)
        _wait_in(b0 + 1, in_b, sem_b)

        @pl.when(b0 + 2 < NB_PER_W)
        def _():
            _start_in(b0 + 2, in_a, sem_a)

        @pl.when(i > 0)
        def _():
            _wait_out(b0 - 1, out_b, osem_b)
        _compute(b0 + 1, in_b, out_b)
        _start_out(b0 + 1, out_b, osem_b)
        return carry

    lax.fori_loop(0, NB_PER_W // 2, pair_body, 0)
    last = NB_PER_W - 1
    _wait_in(last, in_a, sem_a)
    _wait_out(last - 2, out_a, osem_a)
    _compute(last, in_a, out_a)
    _start_out(last, out_a, osem_a)
    _wait_out(last - 1, out_b, osem_b)
    _wait_out(last, out_a, osem_a)


_sc_topk_softmax = functools.partial(
    pl.kernel,
    out_type=jax.ShapeDtypeStruct((R_TOTAL, NPAD), jnp.float32),
    mesh=plsc.VectorSubcoreMesh(core_axis_name="c", subcore_axis_name="s"),
    scratch_types=[
        pltpu.VMEM((BATCH, NPAD), jnp.float32),
        pltpu.VMEM((BATCH, NPAD), jnp.float32),
        pltpu.VMEM((BATCH, NPAD), jnp.float32),
        pltpu.VMEM((BATCH, NPAD), jnp.float32),
        pltpu.SemaphoreType.DMA,
        pltpu.SemaphoreType.DMA,
        pltpu.SemaphoreType.DMA,
        pltpu.SemaphoreType.DMA,
    ],
    compiler_params=pltpu.CompilerParams(needs_layout_passes=False),
)(_sc_body)


def kernel(c_input, weight):
    b, t, n, d = c_input.shape
    xs = c_input.reshape(b * t, n, d)
    g = pl.pallas_call(
        _tc_graph_body,
        grid=(b * t,),
        in_specs=[
            pl.BlockSpec((1, n, d), lambda i: (i, 0, 0)),
            pl.BlockSpec((2, d, weight.shape[2]), lambda i: (0, 0, 0)),
        ],
        out_specs=pl.BlockSpec((NROWPAD, NPAD), lambda i: (i, 0)),
        out_shape=jax.ShapeDtypeStruct((R_TOTAL, NPAD), jnp.float32),
    )(xs, weight)
    out = _sc_topk_softmax(g)
    return out.reshape(b * t, NROWPAD, NPAD)[:, :n, :n].reshape(b, t, n, n)
